# baseline (speedup 1.0000x reference)
import functools

import jax
import jax.numpy as jnp
from jax import lax
from jax.experimental import pallas as pl
from jax.experimental.pallas import tpu as pltpu
from jax.experimental.pallas import tpu_sc as plsc

B = 4096
NNEG = 64
D = 64
NREL = 1000
PW = 72

NC = 2
NS = 16
NW = NC * NS
CHUNK = B // NW
KB = 4
NCH = CHUNK // KB
RPC = KB * NNEG

LN2 = 0.6931471805599453


def _log1p(y):
    x = 1.0 + y
    xb = plsc.bitcast(x, jnp.int32)
    e = ((xb >> 23) & 0xFF) - 127
    mb = (xb & 0x7FFFFF) | 0x3F800000
    m = plsc.bitcast(mb, jnp.float32)
    z = (m - 1.0) / (m + 1.0)
    z2 = z * z
    p = 2.0 * z * (1.0 + z2 * (1.0 / 3.0 + z2 * (1.0 / 5.0 + z2 * (1.0 / 7.0 + z2 * (1.0 / 9.0)))))
    return e.astype(jnp.float32) * LN2 + p


def _sqrt(x):
    xb = plsc.bitcast(x, jnp.int32)
    yb = 0x5F3759DF - (xb >> 1)
    y = plsc.bitcast(yb, jnp.float32)
    for _ in range(3):
        y = y * (1.5 - 0.5 * x * y * y)
    return jnp.where(x > 0.0, x * y, 0.0)


def _sc_kernel(heads, rels, tails, negf, counts, ent, relt, cs,
               pos_o, neg_o,
               hidx_v, tidx_v, ridx_v, negidx_v, cnt_v, cs_v, s_v,
               hrow_v, trow_v, rrow_v, nbuf0, nbuf1, sq_v, pos_sq_v,
               pos_stage, neg_stage, sem, sem0, sem1):
    wid = lax.axis_index("s") * NC + lax.axis_index("c")
    base = wid * CHUNK
    nbufs = (nbuf0, nbuf1)
    sems = (sem0, sem1)

    pltpu.sync_copy(heads.at[pl.ds(base, CHUNK)], hidx_v)
    pltpu.sync_copy(tails.at[pl.ds(base, CHUNK)], tidx_v)
    pltpu.sync_copy(rels.at[pl.ds(base, CHUNK)], ridx_v)
    pltpu.sync_copy(negf.at[pl.ds(base * NNEG, CHUNK * NNEG)], negidx_v)
    pltpu.sync_copy(counts.at[pl.ds(base, CHUNK)], cnt_v)
    pltpu.sync_copy(cs, cs_v)

    c1 = pltpu.async_copy(ent.at[hidx_v], hrow_v, sem)
    c2 = pltpu.async_copy(ent.at[tidx_v], trow_v, sem)
    c3 = pltpu.async_copy(relt.at[ridx_v], rrow_v, sem)
    c1.wait()
    c2.wait()
    c3.wait()

    lane = lax.iota(jnp.int32, 16)
    lane0 = lane == 0

    for g in range(CHUNK // 16):
        rv = ridx_v[pl.ds(g * 16, 16)]
        csg = plsc.load_gather(cs_v, [rv])
        cg = cnt_v[pl.ds(g * 16, 16)]
        s_v[pl.ds(g * 16, 16)] = 1.0 + _log1p(cg * csg)

    def neg_copy(c, p):
        return pltpu.make_async_copy(
            ent.at[negidx_v.at[pl.ds(c * RPC, RPC)]], nbufs[p], sems[p])

    neg_copy(0, 0).start()
    neg_copy(1, 1).start()

    def chunk_pair_body(c2, carry):
        for p in range(2):
            _chunk(c2 * 2 + p, nbufs[p], sems[p], p)
        return carry

    def _chunk(c, nbuf, sem2, p):
        neg_copy(c, p).wait()
        for ii in range(KB):
            i = c * KB + ii
            idx16 = jnp.full((16,), i, jnp.int32)
            s_sp = plsc.load_gather(s_v, [idx16])
            z = [hrow_v[i, pl.ds(g * 16, 16)] + s_sp * rrow_v[i, pl.ds(g * 16, 16)]
                 for g in range(4)]
            acc = None
            for g in range(4):
                d = z[g] - trow_v[i, pl.ds(g * 16, 16)]
                acc = d * d if acc is None else acc + d * d
            plsc.store_scatter(pos_sq_v, [idx16],
                               jnp.broadcast_to(jnp.sum(acc), (16,)), mask=lane0)

            def neg_body(jj, _, _z=z, _ii=ii):
                for u in range(16):
                    r = _ii * NNEG + jj * 16 + u
                    a = None
                    for g in range(4):
                        d = _z[g] - nbuf[r, pl.ds(g * 16, 16)]
                        a = d * d if a is None else a + d * d
                    plsc.store_scatter(sq_v, [jnp.full((16,), r, jnp.int32)],
                                       jnp.broadcast_to(jnp.sum(a), (16,)),
                                       mask=lane0)
                return 0

            lax.fori_loop(0, NNEG // 16, neg_body, 0)

        @pl.when(c + 2 < NCH)
        def _():
            neg_copy(c + 2, p).start()

        for g in range(RPC // 16):
            v = sq_v[pl.ds(g * 16, 16)]
            neg_stage[pl.ds(c * RPC + g * 16, 16)] = _sqrt(v)

    lax.fori_loop(0, NCH // 2, chunk_pair_body, 0)

    for g in range(CHUNK // 16):
        pos_stage[pl.ds(g * 16, 16)] = _sqrt(pos_sq_v[pl.ds(g * 16, 16)])

    pltpu.sync_copy(pos_stage, pos_o.at[pl.ds(base, CHUNK)])
    pltpu.sync_copy(neg_stage, neg_o.at[pl.ds(base * NNEG, CHUNK * NNEG)])


@jax.jit
def _run(heads, relations, tails, neg_flat, counts, ent_table, rel_table, count_scale):
    ent_table = jnp.pad(ent_table, ((0, 0), (0, PW - D)))
    rel_table = jnp.pad(rel_table, ((0, 0), (0, PW - D)))
    mesh = plsc.VectorSubcoreMesh(core_axis_name="c", subcore_axis_name="s")
    k = functools.partial(
        pl.kernel,
        mesh=mesh,
        compiler_params=pltpu.CompilerParams(needs_layout_passes=False,
                                             use_tc_tiling_on_sc=False),
        out_type=[
            jax.ShapeDtypeStruct((B,), jnp.float32),
            jax.ShapeDtypeStruct((B * NNEG,), jnp.float32),
        ],
        scratch_types=[
            pltpu.VMEM((CHUNK,), jnp.int32),
            pltpu.VMEM((CHUNK,), jnp.int32),
            pltpu.VMEM((CHUNK,), jnp.int32),
            pltpu.VMEM((CHUNK * NNEG,), jnp.int32),
            pltpu.VMEM((CHUNK,), jnp.float32),
            pltpu.VMEM((NREL,), jnp.float32),
            pltpu.VMEM((CHUNK,), jnp.float32),
            pltpu.VMEM((CHUNK, PW), jnp.float32),
            pltpu.VMEM((CHUNK, PW), jnp.float32),
            pltpu.VMEM((CHUNK, PW), jnp.float32),
            pltpu.VMEM((RPC, PW), jnp.float32),
            pltpu.VMEM((RPC, PW), jnp.float32),
            pltpu.VMEM((RPC,), jnp.float32),
            pltpu.VMEM((CHUNK,), jnp.float32),
            pltpu.VMEM((CHUNK,), jnp.float32),
            pltpu.VMEM((CHUNK * NNEG,), jnp.float32),
            pltpu.SemaphoreType.DMA,
            pltpu.SemaphoreType.DMA,
            pltpu.SemaphoreType.DMA,
        ],
    )(_sc_kernel)
    return k(heads, relations, tails, neg_flat, counts, ent_table, rel_table, count_scale)


def kernel(heads, relations, tails, neg_tails, counts, ent_table, rel_table, count_scale):
    neg_flat = neg_tails.reshape(-1).astype(jnp.int32)
    pos_flat, neg_score_flat = _run(
        heads.astype(jnp.int32), relations.astype(jnp.int32),
        tails.astype(jnp.int32), neg_flat, counts,
        ent_table, rel_table, count_scale)
    return pos_flat.reshape(B, 1), neg_score_flat.reshape(B, NNEG)


# speedup vs baseline: 1.7487x; 1.7487x over previous
import functools

import jax
import jax.numpy as jnp
from jax import lax
from jax.experimental import pallas as pl
from jax.experimental.pallas import tpu as pltpu
from jax.experimental.pallas import tpu_sc as plsc

B = 4096
NNEG = 64
D = 64
NREL = 1000
PW = 2 * D

NC = 2
NS = 16
NW = NC * NS
CHUNK = B // NW
KB = 2
NCH = CHUNK // KB
RPC = KB * NNEG

LN2 = 0.6931471805599453


def _log1p(y):
    x = 1.0 + y
    xb = plsc.bitcast(x, jnp.int32)
    e = ((xb >> 23) & 0xFF) - 127
    mb = (xb & 0x7FFFFF) | 0x3F800000
    m = plsc.bitcast(mb, jnp.float32)
    z = (m - 1.0) / (m + 1.0)
    z2 = z * z
    p = 2.0 * z * (1.0 + z2 * (1.0 / 3.0 + z2 * (1.0 / 5.0 + z2 * (1.0 / 7.0 + z2 * (1.0 / 9.0)))))
    return e.astype(jnp.float32) * LN2 + p


def _sqrt(x):
    xb = plsc.bitcast(x, jnp.int32)
    yb = 0x5F3759DF - (xb >> 1)
    y = plsc.bitcast(yb, jnp.float32)
    for _ in range(3):
        y = y * (1.5 - 0.5 * x * y * y)
    return jnp.where(x > 0.0, x * y, 0.0)


def _sc_kernel(heads, rels, tails, negf, counts, ent, relt, cs,
               pos_o, neg_o,
               hidx_v, tidx_v, ridx_v, negidx_v, cnt_v, cs_v, s_v,
               hrow_v, trow_v, rrow_v, nbuf0, nbuf1, sq_v, pos_sq_v,
               pos_stage, neg_stage, sem, sem0, sem1):
    wid = lax.axis_index("s") * NC + lax.axis_index("c")
    base = wid * CHUNK
    nbufs = (nbuf0, nbuf1)
    sems = (sem0, sem1)

    pltpu.sync_copy(heads.at[pl.ds(base, CHUNK)], hidx_v)
    pltpu.sync_copy(tails.at[pl.ds(base, CHUNK)], tidx_v)
    pltpu.sync_copy(rels.at[pl.ds(base, CHUNK)], ridx_v)
    pltpu.sync_copy(negf.at[pl.ds(base * NNEG, CHUNK * NNEG)], negidx_v)
    pltpu.sync_copy(counts.at[pl.ds(base, CHUNK)], cnt_v)
    pltpu.sync_copy(cs, cs_v)

    c1 = pltpu.async_copy(ent.at[hidx_v], hrow_v, sem)
    c2 = pltpu.async_copy(ent.at[tidx_v], trow_v, sem)
    c3 = pltpu.async_copy(relt.at[ridx_v], rrow_v, sem)
    c1.wait()
    c2.wait()
    c3.wait()

    lane = lax.iota(jnp.int32, 16)
    lane0 = lane == 0

    for g in range(CHUNK // 16):
        rv = ridx_v[pl.ds(g * 16, 16)]
        csg = plsc.load_gather(cs_v, [rv])
        cg = cnt_v[pl.ds(g * 16, 16)]
        s_v[pl.ds(g * 16, 16)] = 1.0 + _log1p(cg * csg)

    def neg_copy(c, p):
        return pltpu.make_async_copy(
            ent.at[negidx_v.at[pl.ds(c * RPC, RPC)]], nbufs[p], sems[p])

    neg_copy(0, 0).start()
    neg_copy(1, 1).start()

    def chunk_pair_body(c2, carry):
        for p in range(2):
            _chunk(c2 * 2 + p, nbufs[p], sems[p], p)
        return carry

    def _chunk(c, nbuf, sem2, p):
        neg_copy(c, p).wait()
        for ii in range(KB):
            i = c * KB + ii
            idx16 = jnp.full((16,), i, jnp.int32)
            s_sp = plsc.load_gather(s_v, [idx16])
            z = [hrow_v[i, pl.ds(g * 16, 16)] + s_sp * rrow_v[i, pl.ds(g * 16, 16)]
                 for g in range(4)]
            acc = None
            for g in range(4):
                d = z[g] - trow_v[i, pl.ds(g * 16, 16)]
                acc = d * d if acc is None else acc + d * d
            plsc.store_scatter(pos_sq_v, [idx16],
                               jnp.broadcast_to(jnp.sum(acc), (16,)), mask=lane0)

            def neg_body(jj, _, _z=z, _ii=ii):
                for u in range(16):
                    r = _ii * NNEG + jj * 16 + u
                    a = None
                    for g in range(4):
                        d = _z[g] - nbuf[r, pl.ds(g * 16, 16)]
                        a = d * d if a is None else a + d * d
                    plsc.store_scatter(sq_v, [jnp.full((16,), r, jnp.int32)],
                                       jnp.broadcast_to(jnp.sum(a), (16,)),
                                       mask=lane0)
                return 0

            lax.fori_loop(0, NNEG // 16, neg_body, 0)

        @pl.when(c + 2 < NCH)
        def _():
            neg_copy(c + 2, p).start()

        for g in range(RPC // 16):
            v = sq_v[pl.ds(g * 16, 16)]
            neg_stage[pl.ds(c * RPC + g * 16, 16)] = _sqrt(v)

    lax.fori_loop(0, NCH // 2, chunk_pair_body, 0)

    for g in range(CHUNK // 16):
        pos_stage[pl.ds(g * 16, 16)] = _sqrt(pos_sq_v[pl.ds(g * 16, 16)])

    pltpu.sync_copy(pos_stage, pos_o.at[pl.ds(base, CHUNK)])
    pltpu.sync_copy(neg_stage, neg_o.at[pl.ds(base * NNEG, CHUNK * NNEG)])


@jax.jit
def _run(heads, relations, tails, neg_flat, counts, ent_table, rel_table, count_scale):
    ent_table = jnp.pad(ent_table, ((0, 0), (0, PW - D)))
    rel_table = jnp.pad(rel_table, ((0, 0), (0, PW - D)))
    mesh = plsc.VectorSubcoreMesh(core_axis_name="c", subcore_axis_name="s")
    k = functools.partial(
        pl.kernel,
        mesh=mesh,
        compiler_params=pltpu.CompilerParams(needs_layout_passes=False,
                                             use_tc_tiling_on_sc=False),
        out_type=[
            jax.ShapeDtypeStruct((B,), jnp.float32),
            jax.ShapeDtypeStruct((B * NNEG,), jnp.float32),
        ],
        scratch_types=[
            pltpu.VMEM((CHUNK,), jnp.int32),
            pltpu.VMEM((CHUNK,), jnp.int32),
            pltpu.VMEM((CHUNK,), jnp.int32),
            pltpu.VMEM((CHUNK * NNEG,), jnp.int32),
            pltpu.VMEM((CHUNK,), jnp.float32),
            pltpu.VMEM((NREL,), jnp.float32),
            pltpu.VMEM((CHUNK,), jnp.float32),
            pltpu.VMEM((CHUNK, PW), jnp.float32),
            pltpu.VMEM((CHUNK, PW), jnp.float32),
            pltpu.VMEM((CHUNK, PW), jnp.float32),
            pltpu.VMEM((RPC, PW), jnp.float32),
            pltpu.VMEM((RPC, PW), jnp.float32),
            pltpu.VMEM((RPC,), jnp.float32),
            pltpu.VMEM((CHUNK,), jnp.float32),
            pltpu.VMEM((CHUNK,), jnp.float32),
            pltpu.VMEM((CHUNK * NNEG,), jnp.float32),
            pltpu.SemaphoreType.DMA,
            pltpu.SemaphoreType.DMA,
            pltpu.SemaphoreType.DMA,
        ],
    )(_sc_kernel)
    return k(heads, relations, tails, neg_flat, counts, ent_table, rel_table, count_scale)


def kernel(heads, relations, tails, neg_tails, counts, ent_table, rel_table, count_scale):
    neg_flat = neg_tails.reshape(-1).astype(jnp.int32)
    pos_flat, neg_score_flat = _run(
        heads.astype(jnp.int32), relations.astype(jnp.int32),
        tails.astype(jnp.int32), neg_flat, counts,
        ent_table, rel_table, count_scale)
    return pos_flat.reshape(B, 1), neg_score_flat.reshape(B, NNEG)
